# speedup vs baseline: 1.0154x; 1.0154x over previous
import functools

import jax
import jax.numpy as jnp
import numpy as np
from jax import lax
from jax.experimental import pallas as pl
from jax.experimental.pallas import tpu as pltpu
from jax.experimental.pallas import tpu_sc as plsc

B = 4096
K = 128
NIDX = 2 * B
NW = 32
B_PER_W = NIDX // NW
CHUNK = 128
N_CHUNKS = B_PER_W // CHUNK

TILE = 512
T = B // TILE
_PAIRS = np.array(
    [(i, j) for i in range(T) for j in range(i, T)], dtype=np.int32
)
P = len(_PAIRS)


def _sc_gather_body(table_hbm, idx_hbm, out_hbm, idx_v, rows_v, sem):
    wid = lax.axis_index("s") * 2 + lax.axis_index("c")
    base = wid * B_PER_W
    pltpu.sync_copy(idx_hbm.at[wid], idx_v)
    copies = []
    for c in range(N_CHUNKS):
        copies.append(
            pltpu.async_copy(
                table_hbm.at[idx_v.at[c]],
                rows_v.at[pl.ds(c * CHUNK, CHUNK)],
                sem,
            )
        )
    for cp in copies:
        cp.wait()
    pltpu.sync_copy(rows_v, out_hbm.at[pl.ds(base, B_PER_W)])


def _sc_gather(table, idx3):
    mesh = plsc.VectorSubcoreMesh(core_axis_name="c", subcore_axis_name="s")
    fn = functools.partial(
        pl.kernel,
        out_type=jax.ShapeDtypeStruct((NIDX, K), jnp.float32),
        mesh=mesh,
        scratch_types=[
            pltpu.VMEM((N_CHUNKS, CHUNK), jnp.int32),
            pltpu.VMEM((B_PER_W, K), jnp.float32),
            pltpu.SemaphoreType.DMA,
        ],
    )(_sc_gather_body)
    return fn(table, idx3)


def _tc_loss_body(emb_ref, c_ref, e_ref, labc_ref, labr_ref, pairs_ref,
                  out_ref):
    p = pl.program_id(0)
    ti = pairs_ref[p, 0]
    tj = pairs_ref[p, 1]
    ri = pl.multiple_of(ti * TILE, TILE)
    rj = pl.multiple_of(tj * TILE, TILE)

    ei = e_ref[pl.ds(ri, TILE), :]
    ej = e_ref[pl.ds(rj, TILE), :]
    sqrt2 = np.float32(np.sqrt(2.0))
    eib = (ei * sqrt2).astype(jnp.bfloat16)
    ejb = (ej * sqrt2).astype(jnp.bfloat16)
    g2 = lax.dot_general(
        eib, ejb, (((1,), (1,)), ((), ())),
        preferred_element_type=jnp.float32,
    )
    sqi = jnp.sum(ei * ei, axis=1, keepdims=True)
    ej2b = (ej * ej).astype(jnp.bfloat16)
    onesb = jnp.ones((1, K), jnp.bfloat16)
    sqj = lax.dot_general(
        onesb, ej2b, (((1,), (1,)), ((), ())),
        preferred_element_type=jnp.float32,
    )

    hinge = jnp.maximum(0.0, (1.0 + g2) - sqi - sqj)
    mask = labc_ref[pl.ds(ri, TILE), :] != labr_ref[:, pl.ds(rj, TILE)]
    tile_sum = jnp.sum(jnp.where(mask, hinge, 0.0))
    w = jnp.where(ti == tj, 0.0625, 0.125)
    contrib = w * tile_sum

    @pl.when(p == 0)
    def _():
        d = emb_ref[...] - c_ref[...]
        out_ref[0, 0] = jnp.sum(d * d) + contrib

    @pl.when(p != 0)
    def _():
        out_ref[0, 0] += contrib


def _tc_loss(emb, c_rows, e_rows, lab_col, lab_row, pairs):
    return pl.pallas_call(
        _tc_loss_body,
        grid=(P,),
        in_specs=[
            pl.BlockSpec((B, K), lambda p: (0, 0)),
            pl.BlockSpec((B, K), lambda p: (0, 0)),
            pl.BlockSpec((B, K), lambda p: (0, 0)),
            pl.BlockSpec((B, 1), lambda p: (0, 0)),
            pl.BlockSpec((1, B), lambda p: (0, 0)),
            pl.BlockSpec(memory_space=pltpu.SMEM),
        ],
        out_specs=pl.BlockSpec(memory_space=pltpu.SMEM),
        out_shape=jax.ShapeDtypeStruct((1, 1), jnp.float32),
    )(emb, c_rows, e_rows, lab_col, lab_row, pairs)


def kernel(embeddings, labels, table):
    labels = labels.astype(jnp.int32)
    idx_all = jnp.concatenate([labels - 1, labels])
    idx3 = idx_all.reshape(NW, N_CHUNKS, CHUNK)
    rows = _sc_gather(table, idx3)
    c_rows = rows[:B]
    e_rows = rows[B:]
    pairs = jnp.asarray(_PAIRS)
    loss = _tc_loss(
        embeddings, c_rows, e_rows,
        labels.reshape(B, 1), labels.reshape(1, B), pairs,
    )
    return loss[0, 0]


# speedup vs baseline: 1.1202x; 1.1032x over previous
import functools

import jax
import jax.numpy as jnp
import numpy as np
from jax import lax
from jax.experimental import pallas as pl
from jax.experimental.pallas import tpu as pltpu
from jax.experimental.pallas import tpu_sc as plsc

B = 4096
K = 128
NW = 32
ROWS_W = B // NW
LANES = 16

TILE = 512
T = B // TILE
_PAIRS = np.array(
    [(i, j) for i in range(T) for j in range(i, T)], dtype=np.int32
)
P = len(_PAIRS)


def _sc_body(table_hbm, lab_hbm, emb_hbm, e_out_hbm, part_out_hbm,
             lab_v, idxc_v, c_v, e_v, emb_v, part_v, sem_c, sem_e, sem_o):
    wid = lax.axis_index("s") * 2 + lax.axis_index("c")
    base = wid * ROWS_W

    pltpu.sync_copy(lab_hbm.at[wid], lab_v)
    for k in range(ROWS_W // LANES):
        sl = pl.ds(k * LANES, LANES)
        idxc_v[sl] = lab_v[sl] - 1

    gat_c = pltpu.async_copy(table_hbm.at[idxc_v], c_v, sem_c)
    gat_e = pltpu.async_copy(table_hbm.at[lab_v], e_v, sem_e)
    pltpu.sync_copy(emb_hbm.at[pl.ds(base, ROWS_W)], emb_v)

    gat_e.wait()
    put_e = pltpu.async_copy(e_v, e_out_hbm.at[pl.ds(base, ROWS_W)], sem_o)
    gat_c.wait()

    def row_step(r, acc):
        for k in range(K // LANES):
            sl = pl.ds(k * LANES, LANES)
            d = emb_v[r, sl] - c_v[r, sl]
            acc = acc + d * d
        return acc

    acc = lax.fori_loop(0, ROWS_W, row_step, jnp.zeros((LANES,), jnp.float32))
    part_v[...] = acc
    pltpu.sync_copy(part_v, part_out_hbm.at[wid])
    put_e.wait()


def _sc_gather_center(table, lab2, emb):
    mesh = plsc.VectorSubcoreMesh(core_axis_name="c", subcore_axis_name="s")
    fn = functools.partial(
        pl.kernel,
        out_type=(
            jax.ShapeDtypeStruct((B, K), jnp.float32),
            jax.ShapeDtypeStruct((NW, LANES), jnp.float32),
        ),
        mesh=mesh,
        scratch_types=[
            pltpu.VMEM((ROWS_W,), jnp.int32),
            pltpu.VMEM((ROWS_W,), jnp.int32),
            pltpu.VMEM((ROWS_W, K), jnp.float32),
            pltpu.VMEM((ROWS_W, K), jnp.float32),
            pltpu.VMEM((ROWS_W, K), jnp.float32),
            pltpu.VMEM((LANES,), jnp.float32),
            pltpu.SemaphoreType.DMA,
            pltpu.SemaphoreType.DMA,
            pltpu.SemaphoreType.DMA,
        ],
    )(_sc_body)
    return fn(table, lab2, emb)


def _tc_loss_body(e_ref, labc_ref, labr_ref, part_ref, pairs_ref, out_ref):
    p = pl.program_id(0)
    ti = pairs_ref[p, 0]
    tj = pairs_ref[p, 1]
    ri = pl.multiple_of(ti * TILE, TILE)
    rj = pl.multiple_of(tj * TILE, TILE)

    ei = e_ref[pl.ds(ri, TILE), :]
    ej = e_ref[pl.ds(rj, TILE), :]
    sqrt2 = np.float32(np.sqrt(2.0))
    eib = (ei * sqrt2).astype(jnp.bfloat16)
    ejb = (ej * sqrt2).astype(jnp.bfloat16)
    g2 = lax.dot_general(
        eib, ejb, (((1,), (1,)), ((), ())),
        preferred_element_type=jnp.float32,
    )
    sqi = jnp.sum(ei * ei, axis=1, keepdims=True)
    ej2b = (ej * ej).astype(jnp.bfloat16)
    onesb = jnp.ones((1, K), jnp.bfloat16)
    sqj = lax.dot_general(
        onesb, ej2b, (((1,), (1,)), ((), ())),
        preferred_element_type=jnp.float32,
    )

    hinge = jnp.maximum(0.0, (1.0 + g2) - sqi - sqj)
    mask = labc_ref[pl.ds(ri, TILE), :] != labr_ref[:, pl.ds(rj, TILE)]
    tile_sum = jnp.sum(jnp.where(mask, hinge, 0.0))
    w = jnp.where(ti == tj, 0.0625, 0.125)
    contrib = w * tile_sum

    @pl.when(p == 0)
    def _():
        out_ref[0, 0] = jnp.sum(part_ref[...]) + contrib

    @pl.when(p != 0)
    def _():
        out_ref[0, 0] += contrib


def _tc_loss(e_rows, lab_col, lab_row, parts, pairs):
    return pl.pallas_call(
        _tc_loss_body,
        grid=(P,),
        in_specs=[
            pl.BlockSpec((B, K), lambda p: (0, 0)),
            pl.BlockSpec((B, 1), lambda p: (0, 0)),
            pl.BlockSpec((1, B), lambda p: (0, 0)),
            pl.BlockSpec((NW, LANES), lambda p: (0, 0)),
            pl.BlockSpec(memory_space=pltpu.SMEM),
        ],
        out_specs=pl.BlockSpec(memory_space=pltpu.SMEM),
        out_shape=jax.ShapeDtypeStruct((1, 1), jnp.float32),
    )(e_rows, lab_col, lab_row, parts, pairs)


def kernel(embeddings, labels, table):
    labels = labels.astype(jnp.int32)
    lab2 = labels.reshape(NW, ROWS_W)
    e_rows, parts = _sc_gather_center(table, lab2, embeddings)
    pairs = jnp.asarray(_PAIRS)
    loss = _tc_loss(
        e_rows, labels.reshape(B, 1), labels.reshape(1, B), parts, pairs,
    )
    return loss[0, 0]


# speedup vs baseline: 1.2754x; 1.1386x over previous
import functools

import jax
import jax.numpy as jnp
import numpy as np
from jax import lax
from jax.experimental import pallas as pl
from jax.experimental.pallas import tpu as pltpu
from jax.experimental.pallas import tpu_sc as plsc

B = 4096
K = 128
NW = 32
ROWS_W = B // NW
LANES = 16

TILE = 1024
T = B // TILE
_PAIRS = np.array(
    [(i, j) for i in range(T) for j in range(i, T)], dtype=np.int32
)
P = len(_PAIRS)


def _sc_body(table_hbm, lab_hbm, emb_hbm, e_out_hbm, part_out_hbm,
             lab_v, idxc_v, c_v, e_v, emb_v, part_v, sem_c, sem_e, sem_o):
    wid = lax.axis_index("s") * 2 + lax.axis_index("c")
    base = wid * ROWS_W

    pltpu.sync_copy(lab_hbm.at[wid], lab_v)
    for k in range(ROWS_W // LANES):
        sl = pl.ds(k * LANES, LANES)
        idxc_v[sl] = lab_v[sl] - 1

    gat_c = pltpu.async_copy(table_hbm.at[idxc_v], c_v, sem_c)
    gat_e = pltpu.async_copy(table_hbm.at[lab_v], e_v, sem_e)
    pltpu.sync_copy(emb_hbm.at[pl.ds(base, ROWS_W)], emb_v)

    gat_e.wait()
    put_e = pltpu.async_copy(e_v, e_out_hbm.at[pl.ds(base, ROWS_W)], sem_o)
    gat_c.wait()

    def row_step(r, acc):
        for k in range(K // LANES):
            sl = pl.ds(k * LANES, LANES)
            d = emb_v[r, sl] - c_v[r, sl]
            acc = acc + d * d
        return acc

    acc = lax.fori_loop(0, ROWS_W, row_step, jnp.zeros((LANES,), jnp.float32))
    part_v[...] = acc
    pltpu.sync_copy(part_v, part_out_hbm.at[wid])
    put_e.wait()


def _sc_gather_center(table, lab2, emb):
    mesh = plsc.VectorSubcoreMesh(core_axis_name="c", subcore_axis_name="s")
    fn = functools.partial(
        pl.kernel,
        out_type=(
            jax.ShapeDtypeStruct((B, K), jnp.float32),
            jax.ShapeDtypeStruct((NW, LANES), jnp.float32),
        ),
        mesh=mesh,
        scratch_types=[
            pltpu.VMEM((ROWS_W,), jnp.int32),
            pltpu.VMEM((ROWS_W,), jnp.int32),
            pltpu.VMEM((ROWS_W, K), jnp.float32),
            pltpu.VMEM((ROWS_W, K), jnp.float32),
            pltpu.VMEM((ROWS_W, K), jnp.float32),
            pltpu.VMEM((LANES,), jnp.float32),
            pltpu.SemaphoreType.DMA,
            pltpu.SemaphoreType.DMA,
            pltpu.SemaphoreType.DMA,
        ],
    )(_sc_body)
    return fn(table, lab2, emb)


def _tc_loss_body(e_ref, labc_ref, labr_ref, part_ref, pairs_ref, out_ref):
    p = pl.program_id(0)
    ti = pairs_ref[p, 0]
    tj = pairs_ref[p, 1]
    ri = pl.multiple_of(ti * TILE, TILE)
    rj = pl.multiple_of(tj * TILE, TILE)

    ei = e_ref[pl.ds(ri, TILE), :]
    ej = e_ref[pl.ds(rj, TILE), :]
    sqrt2 = np.float32(np.sqrt(2.0))
    eib = (ei * sqrt2).astype(jnp.bfloat16)
    ejb = (ej * sqrt2).astype(jnp.bfloat16)
    g2 = lax.dot_general(
        eib, ejb, (((1,), (1,)), ((), ())),
        preferred_element_type=jnp.float32,
    )
    sqi = jnp.sum(ei * ei, axis=1, keepdims=True)
    ej2b = (ej * ej).astype(jnp.bfloat16)
    onesb = jnp.ones((1, K), jnp.bfloat16)
    sqj = lax.dot_general(
        onesb, ej2b, (((1,), (1,)), ((), ())),
        preferred_element_type=jnp.float32,
    )

    hinge = jnp.maximum(0.0, (1.0 + g2) - sqi - sqj)
    mask = labc_ref[pl.ds(ri, TILE), :] != labr_ref[:, pl.ds(rj, TILE)]
    tile_sum = jnp.sum(jnp.where(mask, hinge, 0.0))
    w = jnp.where(ti == tj, 0.0625, 0.125)
    contrib = w * tile_sum

    @pl.when(p == 0)
    def _():
        out_ref[0, 0] = jnp.sum(part_ref[...]) + contrib

    @pl.when(p != 0)
    def _():
        out_ref[0, 0] += contrib


def _tc_loss(e_rows, lab_col, lab_row, parts, pairs):
    return pl.pallas_call(
        _tc_loss_body,
        grid=(P,),
        in_specs=[
            pl.BlockSpec((B, K), lambda p: (0, 0)),
            pl.BlockSpec((B, 1), lambda p: (0, 0)),
            pl.BlockSpec((1, B), lambda p: (0, 0)),
            pl.BlockSpec((NW, LANES), lambda p: (0, 0)),
            pl.BlockSpec(memory_space=pltpu.SMEM),
        ],
        out_specs=pl.BlockSpec(memory_space=pltpu.SMEM),
        out_shape=jax.ShapeDtypeStruct((1, 1), jnp.float32),
    )(e_rows, lab_col, lab_row, parts, pairs)


def kernel(embeddings, labels, table):
    labels = labels.astype(jnp.int32)
    lab2 = labels.reshape(NW, ROWS_W)
    e_rows, parts = _sc_gather_center(table, lab2, embeddings)
    pairs = jnp.asarray(_PAIRS)
    loss = _tc_loss(
        e_rows, labels.reshape(B, 1), labels.reshape(1, B), parts, pairs,
    )
    return loss[0, 0]
